# baseline (device time: 16270 ns/iter reference)
import jax
import jax.numpy as jnp
from jax import lax
from jax.experimental import pallas as pl
from jax.experimental.pallas import tpu as pltpu

N_DEV = 4
NB = 4


def kernel(x):
    m_per, n_per = x.shape
    rb = m_per // NB
    rows_b = rb // 128

    def body(
        x_hbm, out_hbm,
        xb_ref, e_ref, ob_ref, comm_ref,
        load_sems, store_sems, send_sems, recv_sems,
    ):
        my = lax.axis_index("i")
        p0 = my ^ 1
        p1 = 3 - my

        loads = []
        for b in range(NB):
            cp = pltpu.make_async_copy(
                x_hbm.at[pl.ds(b * rb, rb), :], xb_ref.at[b],
                load_sems.at[b],
            )
            cp.start()
            loads.append(cp)

        barrier_sem = pltpu.get_barrier_semaphore()
        for nbr in [p0, p1]:
            pl.semaphore_signal(
                barrier_sem, inc=1,
                device_id=(nbr,), device_id_type=pl.DeviceIdType.MESH,
            )
        pl.semaphore_wait(barrier_sem, 2)

        def partial(b):
            loads[b].wait()
            xv = xb_ref[b]
            m = jnp.max(xv, axis=1, keepdims=True)
            e = jnp.exp(xv - m)
            e_ref[pl.ds(b * rb, rb), :] = e.astype(jnp.bfloat16)
            s = jnp.sum(e, axis=1, keepdims=True)
            comm_ref[b, 0, 0:rows_b, :] = m.reshape(rows_b, 128)
            comm_ref[b, 0, rows_b:2 * rows_b, :] = s.reshape(rows_b, 128)

        def start_ex(b, stage, partner):
            src, dst = (0, 1) if stage == 0 else (2, 3)
            rdma = pltpu.make_async_remote_copy(
                src_ref=comm_ref.at[b, src],
                dst_ref=comm_ref.at[b, dst],
                send_sem=send_sems.at[2 * b + stage],
                recv_sem=recv_sems.at[2 * b + stage],
                device_id=(partner,),
                device_id_type=pl.DeviceIdType.MESH,
            )
            rdma.start()
            return rdma

        def pair(slot_m, slot_s):
            return slot_m, slot_s

        def combine0(b):
            m0 = comm_ref[b, 0, 0:rows_b, :]
            s0 = comm_ref[b, 0, rows_b:2 * rows_b, :]
            m1 = comm_ref[b, 1, 0:rows_b, :]
            s1 = comm_ref[b, 1, rows_b:2 * rows_b, :]
            M01 = jnp.maximum(m0, m1)
            S01 = s0 * jnp.exp(m0 - M01) + s1 * jnp.exp(m1 - M01)
            comm_ref[b, 2, 0:rows_b, :] = M01
            comm_ref[b, 2, rows_b:2 * rows_b, :] = S01

        def final(b):
            m2 = comm_ref[b, 2, 0:rows_b, :]
            s2 = comm_ref[b, 2, rows_b:2 * rows_b, :]
            m3 = comm_ref[b, 3, 0:rows_b, :]
            s3 = comm_ref[b, 3, rows_b:2 * rows_b, :]
            M = jnp.maximum(m2, m3)
            S = s2 * jnp.exp(m2 - M) + s3 * jnp.exp(m3 - M)
            m_own = comm_ref[b, 0, 0:rows_b, :]
            scale = (jnp.exp(m_own - M) / S).astype(jnp.bfloat16)
            e3 = e_ref[pl.ds(b * rb, rb), :].reshape(rows_b, 128, n_per)
            scale_b = lax.broadcast_in_dim(
                scale, (rows_b, 128, n_per), (0, 1)
            )
            ob_ref[b] = (e3 * scale_b).reshape(rb, n_per)
            st = pltpu.make_async_copy(
                ob_ref.at[b], out_hbm.at[pl.ds(b * rb, rb), :],
                store_sems.at[b],
            )
            st.start()
            return st

        ex = {}
        partial(0)
        ex[(0, 0)] = start_ex(0, 0, p0)
        partial(1)
        ex[(1, 0)] = start_ex(1, 0, p0)
        ex[(0, 0)].wait()
        combine0(0)
        ex[(0, 1)] = start_ex(0, 1, p1)
        partial(2)
        ex[(2, 0)] = start_ex(2, 0, p0)
        ex[(1, 0)].wait()
        combine0(1)
        ex[(1, 1)] = start_ex(1, 1, p1)
        partial(3)
        ex[(3, 0)] = start_ex(3, 0, p0)
        ex[(0, 1)].wait()
        stores = [final(0)]
        ex[(2, 0)].wait()
        combine0(2)
        ex[(2, 1)] = start_ex(2, 1, p1)
        ex[(1, 1)].wait()
        stores.append(final(1))
        ex[(3, 0)].wait()
        combine0(3)
        ex[(3, 1)] = start_ex(3, 1, p1)
        ex[(2, 1)].wait()
        stores.append(final(2))
        ex[(3, 1)].wait()
        stores.append(final(3))
        for st in stores:
            st.wait()

    return pl.pallas_call(
        body,
        out_shape=jax.ShapeDtypeStruct((m_per, n_per), jnp.bfloat16),
        in_specs=[pl.BlockSpec(memory_space=pl.ANY)],
        out_specs=pl.BlockSpec(memory_space=pl.ANY),
        scratch_shapes=[
            pltpu.VMEM((NB, rb, n_per), jnp.float32),
            pltpu.VMEM((m_per, n_per), jnp.bfloat16),
            pltpu.VMEM((NB, rb, n_per), jnp.bfloat16),
            pltpu.VMEM((NB, 4, 2 * rows_b, 128), jnp.float32),
            pltpu.SemaphoreType.DMA((NB,)),
            pltpu.SemaphoreType.DMA((NB,)),
            pltpu.SemaphoreType.DMA((2 * NB,)),
            pltpu.SemaphoreType.DMA((2 * NB,)),
        ],
        compiler_params=pltpu.CompilerParams(collective_id=0),
    )(x)


# device time: 12145 ns/iter; 1.3396x vs baseline; 1.3396x over previous
import jax
import jax.numpy as jnp
from jax import lax
from jax.experimental import pallas as pl
from jax.experimental.pallas import tpu as pltpu

N_DEV = 4
NB = 2


def kernel(x):
    m_per, n_per = x.shape
    rb = m_per // NB
    rows_b = rb // 128

    def body(x_ref, out_ref, e_ref, comm_ref, send_sems, recv_sems):
        my = lax.axis_index("i")

        barrier_sem = pltpu.get_barrier_semaphore()
        for o in range(1, N_DEV):
            pl.semaphore_signal(
                barrier_sem, inc=1,
                device_id=((my + o) % N_DEV,),
                device_id_type=pl.DeviceIdType.MESH,
            )

        def partial(b):
            xv = x_ref[pl.ds(b * rb, rb), :]
            m = jnp.max(xv, axis=1, keepdims=True)
            e = jnp.exp(xv - m)
            e_ref[pl.ds(b * rb, rb), :] = e.astype(jnp.bfloat16)
            s = jnp.sum(e, axis=1, keepdims=True)
            comm_ref[b, 0, 0:rows_b, :] = m.reshape(rows_b, 128)
            comm_ref[b, 0, rows_b:2 * rows_b, :] = s.reshape(rows_b, 128)

        def start_a2a(b):
            rdmas = []
            for o in range(1, N_DEV):
                rdma = pltpu.make_async_remote_copy(
                    src_ref=comm_ref.at[b, 0],
                    dst_ref=comm_ref.at[b, o],
                    send_sem=send_sems.at[3 * b + o - 1],
                    recv_sem=recv_sems.at[3 * b + o - 1],
                    device_id=((my + o) % N_DEV,),
                    device_id_type=pl.DeviceIdType.MESH,
                )
                rdma.start()
                rdmas.append(rdma)
            return rdmas

        def final(b, rdmas):
            for rdma in rdmas:
                rdma.wait()
            M = comm_ref[b, 0, 0:rows_b, :]
            for k in range(1, N_DEV):
                M = jnp.maximum(M, comm_ref[b, k, 0:rows_b, :])
            S = jnp.zeros_like(M)
            for k in range(N_DEV):
                S = S + comm_ref[b, k, rows_b:2 * rows_b, :] * jnp.exp(
                    comm_ref[b, k, 0:rows_b, :] - M
                )
            m0 = comm_ref[b, 0, 0:rows_b, :]
            scale = (jnp.exp(m0 - M) / S).astype(jnp.bfloat16)
            e3 = e_ref[pl.ds(b * rb, rb), :].reshape(rows_b, 128, n_per)
            scale_b = lax.broadcast_in_dim(
                scale, (rows_b, 128, n_per), (0, 1)
            )
            out_ref[pl.ds(b * rb, rb), :] = (e3 * scale_b).reshape(
                rb, n_per
            )

        rdmas = []
        for b in range(NB):
            partial(b)
            if b == 0:
                pl.semaphore_wait(barrier_sem, N_DEV - 1)
            rdmas.append(start_a2a(b))
        for b in range(NB):
            final(b, rdmas[b])

    return pl.pallas_call(
        body,
        out_shape=jax.ShapeDtypeStruct((m_per, n_per), jnp.bfloat16),
        in_specs=[pl.BlockSpec(memory_space=pltpu.VMEM)],
        out_specs=pl.BlockSpec(memory_space=pltpu.VMEM),
        scratch_shapes=[
            pltpu.VMEM((m_per, n_per), jnp.bfloat16),
            pltpu.VMEM((NB, N_DEV, 2 * rows_b, 128), jnp.float32),
            pltpu.SemaphoreType.DMA((3 * NB,)),
            pltpu.SemaphoreType.DMA((3 * NB,)),
        ],
        compiler_params=pltpu.CompilerParams(collective_id=0),
    )(x)
